# baseline (speedup 1.0000x reference)
import functools

import jax
import jax.numpy as jnp
from jax.experimental import pallas as pl

_F32 = jnp.float32
_BF16 = jnp.bfloat16


def _attn_body(q_ref, k_ref, v_ref, wo_ref, bo_ref, out_ref, *, chunk):
    h = pl.program_id(1)
    kk = k_ref[0]
    vv = v_ref[0]
    wo = wo_ref[0]
    n_rows = q_ref.shape[1]
    for ci in range(n_rows // chunk):
        rows = slice(ci * chunk, (ci + 1) * chunk)
        q = q_ref[0, rows, :]
        s = jax.lax.dot_general(q, kk, (((1,), (1,)), ((), ())),
                                preferred_element_type=_F32)
        m = jnp.max(s, axis=-1, keepdims=True)
        e = jnp.exp(s - m).astype(_BF16)
        o = jnp.dot(e, vv, preferred_element_type=_F32)
        o = o * (1.0 / jnp.sum(e.astype(_F32), axis=-1, keepdims=True))
        contrib = jnp.dot(o, wo, preferred_element_type=_F32)

        @pl.when(h == 0)
        def _(contrib=contrib, rows=rows):
            out_ref[rows, :] = contrib + bo_ref[...]

        @pl.when(h != 0)
        def _(contrib=contrib, rows=rows):
            out_ref[rows, :] = out_ref[rows, :] + contrib


def kernel(pos, feat, mask, k, pos_lambda, qkv_w, qkv_b, pos_mlp_w,
           pos_mlp_b, proj_w, proj_b):
    b, n, c = feat.shape
    d = pos.shape[2]
    nh = pos_mlp_w.shape[0]
    c_ = c // nh
    qd = c_ // 2
    vd = 2 * c_
    AUG = 64
    scale = (c_ ** -0.5) * k

    feat2 = feat.reshape(n, c)
    pos2 = pos.reshape(n, d).astype(_F32)
    mask2 = mask.reshape(n, 1)

    w6 = qkv_w.reshape(nh, 6, qd, c)
    b6 = qkv_b.reshape(nh, 6, qd)
    wq = jnp.swapaxes(w6[:, 0], 1, 2)
    wk = jnp.swapaxes(w6[:, 1], 1, 2)
    wv = jnp.swapaxes(w6[:, 2:].reshape(nh, vd, c), 1, 2)
    bq, bk = b6[:, 0], b6[:, 1]
    bv = b6[:, 2:].reshape(nh, 1, vd)

    wq_aug = jnp.zeros((nh, c, AUG), _F32).at[:, :, :qd].set(wq * scale)
    bq_aug = (jnp.zeros((nh, 1, AUG), _F32).at[:, 0, :qd].set(bq * scale)
              .at[:, 0, qd].set(1.0))
    wk_aug = jnp.zeros((nh, c, AUG), _F32).at[:, :, :qd].set(wk)
    bk_aug = (jnp.zeros((nh, 1, AUG), _F32).at[:, 0, :qd].set(bk)
              .at[:, 0, qd].set(-100.0))
    wpos_aug = jnp.zeros((nh, d, AUG), _F32).at[:, :, qd].set(pos_mlp_w[:, :, 0])
    sel_aug = jnp.zeros((nh, 1, AUG), _F32).at[:, 0, qd].set(1.0)

    wo = jnp.transpose(proj_w.reshape(c, nh, vd), (1, 2, 0))
    bo = proj_b.reshape(1, c)

    qkv_specs = [
        pl.BlockSpec((n, c), lambda hh: (0, 0)),
        pl.BlockSpec((n, d), lambda hh: (0, 0)),
        pl.BlockSpec((n, 1), lambda hh: (0, 0)),
        pl.BlockSpec((1, c, AUG), lambda hh: (hh, 0, 0)),
        pl.BlockSpec((1, 1, AUG), lambda hh: (hh, 0, 0)),
        pl.BlockSpec((1, c, AUG), lambda hh: (hh, 0, 0)),
        pl.BlockSpec((1, 1, AUG), lambda hh: (hh, 0, 0)),
        pl.BlockSpec((1, d, AUG), lambda hh: (hh, 0, 0)),
        pl.BlockSpec((1, 1, AUG), lambda hh: (hh, 0, 0)),
        pl.BlockSpec((1, c, vd), lambda hh: (hh, 0, 0)),
        pl.BlockSpec((1, 1, vd), lambda hh: (hh, 0, 0)),
    ]
    q_a, k_a, v_a = pl.pallas_call(
        _proj_body_impl,
        grid=(nh,),
        in_specs=qkv_specs,
        out_specs=[
            pl.BlockSpec((1, n, AUG), lambda hh: (hh, 0, 0)),
            pl.BlockSpec((1, n, AUG), lambda hh: (hh, 0, 0)),
            pl.BlockSpec((1, n, vd), lambda hh: (hh, 0, 0)),
        ],
        out_shape=[
            jax.ShapeDtypeStruct((nh, n, AUG), _F32),
            jax.ShapeDtypeStruct((nh, n, AUG), _F32),
            jax.ShapeDtypeStruct((nh, n, vd), _BF16),
        ],
    )(feat2, pos2, mask2, wq_aug, bq_aug, wk_aug, bk_aug, wpos_aug, sel_aug,
      wv, bv)

    R = 2048
    nr = n // R
    out2 = pl.pallas_call(
        functools.partial(_attn_body, chunk=2048),
        grid=(nr, nh),
        in_specs=[
            pl.BlockSpec((1, R, AUG), lambda r, hh: (hh, r, 0)),
            pl.BlockSpec((1, n, AUG), lambda r, hh: (hh, 0, 0)),
            pl.BlockSpec((1, n, vd), lambda r, hh: (hh, 0, 0)),
            pl.BlockSpec((1, vd, c), lambda r, hh: (hh, 0, 0)),
            pl.BlockSpec((1, c), lambda r, hh: (0, 0)),
        ],
        out_specs=pl.BlockSpec((R, c), lambda r, hh: (r, 0)),
        out_shape=jax.ShapeDtypeStruct((n, c), _F32),
    )(q_a, k_a, v_a, wo, bo)

    return out2.reshape(b, n, c)


def _proj_body_impl(feat_ref, pos_ref, mask_ref, wq_ref, bq_ref, wk_ref,
                    bk_ref, wpos_ref, sel_ref, wv_ref, bv_ref,
                    q_out, k_out, v_out):
    f = feat_ref[...]
    q_out[0] = jnp.dot(f, wq_ref[0], preferred_element_type=_F32) + bq_ref[0]
    p = pos_ref[...]
    mx = jnp.max(p, axis=0, keepdims=True)
    pn = p / jnp.maximum(mx, 1e-30)
    w_pos = wpos_ref[0]
    d = w_pos.shape[0]
    bias_cols = pn[:, 0:1] * w_pos[0:1, :]
    for i in range(1, d):
        bias_cols = bias_cols + pn[:, i:i + 1] * w_pos[i:i + 1, :]
    bias_cols = bias_cols + (mask_ref[...] * 100.0) * sel_ref[0]
    k_out[0] = (jnp.dot(f, wk_ref[0], preferred_element_type=_F32)
                + bias_cols + bk_ref[0])
    v_out[0] = (jnp.dot(f, wv_ref[0], preferred_element_type=_F32)
                + bv_ref[0]).astype(_BF16)


# speedup vs baseline: 1.0492x; 1.0492x over previous
import jax
import jax.numpy as jnp
from jax.experimental import pallas as pl

_F32 = jnp.float32


def _fused_body(feat_ref, pos_ref, mask_ref, wq_ref, bq_ref, wk_ref, bk_ref,
                wpos_ref, sel_ref, wv_ref, bv_ref, wo_ref, bo_ref, out_ref):
    h = pl.program_id(0)
    f = feat_ref[...]
    q = jnp.dot(f, wq_ref[0], preferred_element_type=_F32) + bq_ref[0]
    p = pos_ref[...]
    mx = jnp.max(p, axis=0, keepdims=True)
    pn = p / jnp.maximum(mx, 1e-30)
    w_pos = wpos_ref[0]
    d = w_pos.shape[0]
    bias_cols = pn[:, 0:1] * w_pos[0:1, :]
    for i in range(1, d):
        bias_cols = bias_cols + pn[:, i:i + 1] * w_pos[i:i + 1, :]
    bias_cols = bias_cols + (mask_ref[...] * 100.0) * sel_ref[0]
    kk = (jnp.dot(f, wk_ref[0], preferred_element_type=_F32)
          + bias_cols + bk_ref[0])
    vv = jnp.dot(f, wv_ref[0], preferred_element_type=_F32) + bv_ref[0]

    s = jax.lax.dot_general(q, kk, (((1,), (1,)), ((), ())),
                            preferred_element_type=_F32)
    m = jnp.max(s, axis=-1, keepdims=True)
    e = jnp.exp(s - m)
    o = jnp.dot(e, vv, preferred_element_type=_F32)
    o = o * (1.0 / jnp.sum(e, axis=-1, keepdims=True))
    contrib = jnp.dot(o, wo_ref[0], preferred_element_type=_F32)

    @pl.when(h == 0)
    def _():
        out_ref[...] = contrib + bo_ref[...]

    @pl.when(h != 0)
    def _():
        out_ref[...] = out_ref[...] + contrib


def kernel(pos, feat, mask, k, pos_lambda, qkv_w, qkv_b, pos_mlp_w,
           pos_mlp_b, proj_w, proj_b):
    b, n, c = feat.shape
    d = pos.shape[2]
    nh = pos_mlp_w.shape[0]
    c_ = c // nh
    qd = c_ // 2
    vd = 2 * c_
    AUG = 64
    scale = (c_ ** -0.5) * k

    feat2 = feat.reshape(n, c)
    pos2 = pos.reshape(n, d).astype(_F32)
    mask2 = mask.reshape(n, 1)

    w6 = qkv_w.reshape(nh, 6, qd, c)
    b6 = qkv_b.reshape(nh, 6, qd)
    wq = jnp.swapaxes(w6[:, 0], 1, 2)
    wk = jnp.swapaxes(w6[:, 1], 1, 2)
    wv = jnp.swapaxes(w6[:, 2:].reshape(nh, vd, c), 1, 2)
    bq, bk = b6[:, 0], b6[:, 1]
    bv = b6[:, 2:].reshape(nh, 1, vd)

    wq_aug = jnp.zeros((nh, c, AUG), _F32).at[:, :, :qd].set(wq * scale)
    bq_aug = (jnp.zeros((nh, 1, AUG), _F32).at[:, 0, :qd].set(bq * scale)
              .at[:, 0, qd].set(1.0))
    wk_aug = jnp.zeros((nh, c, AUG), _F32).at[:, :, :qd].set(wk)
    bk_aug = (jnp.zeros((nh, 1, AUG), _F32).at[:, 0, :qd].set(bk)
              .at[:, 0, qd].set(-100.0))
    wpos_aug = jnp.zeros((nh, d, AUG), _F32).at[:, :, qd].set(pos_mlp_w[:, :, 0])
    sel_aug = jnp.zeros((nh, 1, AUG), _F32).at[:, 0, qd].set(1.0)

    wo = jnp.transpose(proj_w.reshape(c, nh, vd), (1, 2, 0))
    bo = proj_b.reshape(1, c)

    out2 = pl.pallas_call(
        _fused_body,
        grid=(nh,),
        in_specs=[
            pl.BlockSpec((n, c), lambda hh: (0, 0)),
            pl.BlockSpec((n, d), lambda hh: (0, 0)),
            pl.BlockSpec((n, 1), lambda hh: (0, 0)),
            pl.BlockSpec((1, c, AUG), lambda hh: (hh, 0, 0)),
            pl.BlockSpec((1, 1, AUG), lambda hh: (hh, 0, 0)),
            pl.BlockSpec((1, c, AUG), lambda hh: (hh, 0, 0)),
            pl.BlockSpec((1, 1, AUG), lambda hh: (hh, 0, 0)),
            pl.BlockSpec((1, d, AUG), lambda hh: (hh, 0, 0)),
            pl.BlockSpec((1, 1, AUG), lambda hh: (hh, 0, 0)),
            pl.BlockSpec((1, c, vd), lambda hh: (hh, 0, 0)),
            pl.BlockSpec((1, 1, vd), lambda hh: (hh, 0, 0)),
            pl.BlockSpec((1, vd, c), lambda hh: (hh, 0, 0)),
            pl.BlockSpec((1, c), lambda hh: (0, 0)),
        ],
        out_specs=pl.BlockSpec((n, c), lambda hh: (0, 0)),
        out_shape=jax.ShapeDtypeStruct((n, c), _F32),
    )(feat2, pos2, mask2, wq_aug, bq_aug, wk_aug, bk_aug, wpos_aug, sel_aug,
      wv, bv, wo, bo)

    return out2.reshape(b, n, c)


# speedup vs baseline: 1.3454x; 1.2823x over previous
import jax
import jax.numpy as jnp
from jax.experimental import pallas as pl

_F32 = jnp.float32


def _fused_body(feat_ref, pos_ref, mask_ref, wq_ref, bq_ref, wk_ref, bk_ref,
                wpos_ref, sel_ref, wv_ref, bv_ref, wo_ref, bo_ref, out_ref):
    h = pl.program_id(0)
    f = feat_ref[...]
    n = f.shape[0]
    p = pos_ref[...]
    mx = jnp.max(p, axis=0, keepdims=True)
    pn = p / jnp.maximum(mx, 1e-30)
    d = p.shape[1]
    mask_term = mask_ref[...] * 100.0

    def head(i):
        q = jnp.dot(f, wq_ref[i], preferred_element_type=_F32) + bq_ref[i]
        w_pos = wpos_ref[i]
        bias_cols = pn[:, 0:1] * w_pos[0:1, :]
        for j in range(1, d):
            bias_cols = bias_cols + pn[:, j:j + 1] * w_pos[j:j + 1, :]
        bias_cols = bias_cols + mask_term * sel_ref[i]
        kk = (jnp.dot(f, wk_ref[i], preferred_element_type=_F32)
              + bias_cols + bk_ref[i])
        vv = jnp.dot(f, wv_ref[i], preferred_element_type=_F32) + bv_ref[i]
        return q, kk, vv

    q0, k0, v0 = head(0)
    q1, k1, v1 = head(1)

    nch = 2
    ch = n // nch
    for ci in range(nch):
        rows = slice(ci * ch, (ci + 1) * ch)
        s0 = jax.lax.dot_general(q0[rows], k0, (((1,), (1,)), ((), ())),
                                 preferred_element_type=_F32)
        s1 = jax.lax.dot_general(q1[rows], k1, (((1,), (1,)), ((), ())),
                                 preferred_element_type=_F32)
        e0 = jnp.exp(s0 - jnp.max(s0, axis=-1, keepdims=True))
        e1 = jnp.exp(s1 - jnp.max(s1, axis=-1, keepdims=True))
        o0 = jnp.dot(e0, v0, preferred_element_type=_F32)
        o1 = jnp.dot(e1, v1, preferred_element_type=_F32)
        o0 = o0 * (1.0 / jnp.sum(e0, axis=-1, keepdims=True))
        o1 = o1 * (1.0 / jnp.sum(e1, axis=-1, keepdims=True))
        contrib = (jnp.dot(o0, wo_ref[0], preferred_element_type=_F32)
                   + jnp.dot(o1, wo_ref[1], preferred_element_type=_F32))

        @pl.when(h == 0)
        def _(contrib=contrib, rows=rows):
            out_ref[rows, :] = contrib + bo_ref[...]

        @pl.when(h != 0)
        def _(contrib=contrib, rows=rows):
            out_ref[rows, :] = out_ref[rows, :] + contrib


def kernel(pos, feat, mask, k, pos_lambda, qkv_w, qkv_b, pos_mlp_w,
           pos_mlp_b, proj_w, proj_b):
    b, n, c = feat.shape
    d = pos.shape[2]
    nh = pos_mlp_w.shape[0]
    c_ = c // nh
    qd = c_ // 2
    vd = 2 * c_
    AUG = 64
    scale = (c_ ** -0.5) * k

    feat2 = feat.reshape(n, c)
    pos2 = pos.reshape(n, d).astype(_F32)
    mask2 = mask.reshape(n, 1)

    w6 = qkv_w.reshape(nh, 6, qd, c)
    b6 = qkv_b.reshape(nh, 6, qd)
    wq = jnp.swapaxes(w6[:, 0], 1, 2)
    wk = jnp.swapaxes(w6[:, 1], 1, 2)
    wv = jnp.swapaxes(w6[:, 2:].reshape(nh, vd, c), 1, 2)
    bq, bk = b6[:, 0], b6[:, 1]
    bv = b6[:, 2:].reshape(nh, 1, vd)

    wq_aug = jnp.zeros((nh, c, AUG), _F32).at[:, :, :qd].set(wq * scale)
    bq_aug = (jnp.zeros((nh, 1, AUG), _F32).at[:, 0, :qd].set(bq * scale)
              .at[:, 0, qd].set(1.0))
    wk_aug = jnp.zeros((nh, c, AUG), _F32).at[:, :, :qd].set(wk)
    bk_aug = (jnp.zeros((nh, 1, AUG), _F32).at[:, 0, :qd].set(bk)
              .at[:, 0, qd].set(-100.0))
    wpos_aug = jnp.zeros((nh, d, AUG), _F32).at[:, :, qd].set(pos_mlp_w[:, :, 0])
    sel_aug = jnp.zeros((nh, 1, AUG), _F32).at[:, 0, qd].set(1.0)

    wo = jnp.transpose(proj_w.reshape(c, nh, vd), (1, 2, 0))
    bo = proj_b.reshape(1, c)

    out2 = pl.pallas_call(
        _fused_body,
        grid=(nh // 2,),
        in_specs=[
            pl.BlockSpec((n, c), lambda hh: (0, 0)),
            pl.BlockSpec((n, d), lambda hh: (0, 0)),
            pl.BlockSpec((n, 1), lambda hh: (0, 0)),
            pl.BlockSpec((2, c, AUG), lambda hh: (hh, 0, 0)),
            pl.BlockSpec((2, 1, AUG), lambda hh: (hh, 0, 0)),
            pl.BlockSpec((2, c, AUG), lambda hh: (hh, 0, 0)),
            pl.BlockSpec((2, 1, AUG), lambda hh: (hh, 0, 0)),
            pl.BlockSpec((2, d, AUG), lambda hh: (hh, 0, 0)),
            pl.BlockSpec((2, 1, AUG), lambda hh: (hh, 0, 0)),
            pl.BlockSpec((2, c, vd), lambda hh: (hh, 0, 0)),
            pl.BlockSpec((2, 1, vd), lambda hh: (hh, 0, 0)),
            pl.BlockSpec((2, vd, c), lambda hh: (hh, 0, 0)),
            pl.BlockSpec((1, c), lambda hh: (0, 0)),
        ],
        out_specs=pl.BlockSpec((n, c), lambda hh: (0, 0)),
        out_shape=jax.ShapeDtypeStruct((n, c), _F32),
    )(feat2, pos2, mask2, wq_aug, bq_aug, wk_aug, bk_aug, wpos_aug, sel_aug,
      wv, bv, wo, bo)

    return out2.reshape(b, n, c)


# speedup vs baseline: 1.4094x; 1.0475x over previous
import jax
import jax.numpy as jnp
from jax.experimental import pallas as pl

_F32 = jnp.float32


def _fused_body(feat_ref, pos_ref, mask_ref, wq_ref, bq_ref, wk_ref, bk_ref,
                wpos_ref, sel_ref, wv_ref, bv_ref, wo_ref, bo_ref, out_ref):
    h = pl.program_id(0)
    f = feat_ref[...]
    n = f.shape[0]
    p = pos_ref[...]
    mx = jnp.max(p, axis=0, keepdims=True)
    pn = p / jnp.maximum(mx, 1e-30)
    d = p.shape[1]
    mask_term = mask_ref[...] * 100.0

    def head(i):
        q = jnp.dot(f, wq_ref[i], preferred_element_type=_F32) + bq_ref[i]
        w_pos = wpos_ref[i]
        bias_cols = pn[:, 0:1] * w_pos[0:1, :]
        for j in range(1, d):
            bias_cols = bias_cols + pn[:, j:j + 1] * w_pos[j:j + 1, :]
        bias_cols = bias_cols + mask_term * sel_ref[i]
        kk = (jnp.dot(f, wk_ref[i], preferred_element_type=_F32)
              + bias_cols + bk_ref[i])
        vv = jnp.dot(f, wv_ref[i], preferred_element_type=_F32) + bv_ref[i]
        return q, kk, vv

    q0, k0, v0 = head(0)
    q1, k1, v1 = head(1)
    q2, k2, v2 = head(2)

    nch = 2
    ch = n // nch
    for ci in range(nch):
        rows = slice(ci * ch, (ci + 1) * ch)
        s0 = jax.lax.dot_general(q0[rows], k0, (((1,), (1,)), ((), ())),
                                 preferred_element_type=_F32)
        s1 = jax.lax.dot_general(q1[rows], k1, (((1,), (1,)), ((), ())),
                                 preferred_element_type=_F32)
        s2 = jax.lax.dot_general(q2[rows], k2, (((1,), (1,)), ((), ())),
                                 preferred_element_type=_F32)
        e0 = jnp.exp(s0 - jnp.max(s0, axis=-1, keepdims=True))
        e1 = jnp.exp(s1 - jnp.max(s1, axis=-1, keepdims=True))
        e2 = jnp.exp(s2 - jnp.max(s2, axis=-1, keepdims=True))
        o0 = jnp.dot(e0, v0, preferred_element_type=_F32)
        o1 = jnp.dot(e1, v1, preferred_element_type=_F32)
        o2 = jnp.dot(e2, v2, preferred_element_type=_F32)
        o0 = o0 * (1.0 / jnp.sum(e0, axis=-1, keepdims=True))
        o1 = o1 * (1.0 / jnp.sum(e1, axis=-1, keepdims=True))
        o2 = o2 * (1.0 / jnp.sum(e2, axis=-1, keepdims=True))
        contrib = (jnp.dot(o0, wo_ref[0], preferred_element_type=_F32)
                   + jnp.dot(o1, wo_ref[1], preferred_element_type=_F32)
                   + jnp.dot(o2, wo_ref[2], preferred_element_type=_F32))

        @pl.when(h == 0)
        def _(contrib=contrib, rows=rows):
            out_ref[rows, :] = contrib + bo_ref[...]

        @pl.when(h != 0)
        def _(contrib=contrib, rows=rows):
            out_ref[rows, :] = out_ref[rows, :] + contrib


def kernel(pos, feat, mask, k, pos_lambda, qkv_w, qkv_b, pos_mlp_w,
           pos_mlp_b, proj_w, proj_b):
    b, n, c = feat.shape
    d = pos.shape[2]
    nh = pos_mlp_w.shape[0]
    c_ = c // nh
    qd = c_ // 2
    vd = 2 * c_
    AUG = 64
    scale = (c_ ** -0.5) * k

    feat2 = feat.reshape(n, c)
    pos2 = pos.reshape(n, d).astype(_F32)
    mask2 = mask.reshape(n, 1)

    w6 = qkv_w.reshape(nh, 6, qd, c)
    b6 = qkv_b.reshape(nh, 6, qd)
    wq = jnp.swapaxes(w6[:, 0], 1, 2)
    wk = jnp.swapaxes(w6[:, 1], 1, 2)
    wv = jnp.swapaxes(w6[:, 2:].reshape(nh, vd, c), 1, 2)
    bq, bk = b6[:, 0], b6[:, 1]
    bv = b6[:, 2:].reshape(nh, 1, vd)

    wq_aug = jnp.zeros((nh, c, AUG), _F32).at[:, :, :qd].set(wq * scale)
    bq_aug = (jnp.zeros((nh, 1, AUG), _F32).at[:, 0, :qd].set(bq * scale)
              .at[:, 0, qd].set(1.0))
    wk_aug = jnp.zeros((nh, c, AUG), _F32).at[:, :, :qd].set(wk)
    bk_aug = (jnp.zeros((nh, 1, AUG), _F32).at[:, 0, :qd].set(bk)
              .at[:, 0, qd].set(-100.0))
    wpos_aug = jnp.zeros((nh, d, AUG), _F32).at[:, :, qd].set(pos_mlp_w[:, :, 0])
    sel_aug = jnp.zeros((nh, 1, AUG), _F32).at[:, 0, qd].set(1.0)

    wo = jnp.transpose(proj_w.reshape(c, nh, vd), (1, 2, 0))
    bo = proj_b.reshape(1, c)

    out2 = pl.pallas_call(
        _fused_body,
        grid=(nh // 3,),
        in_specs=[
            pl.BlockSpec((n, c), lambda hh: (0, 0)),
            pl.BlockSpec((n, d), lambda hh: (0, 0)),
            pl.BlockSpec((n, 1), lambda hh: (0, 0)),
            pl.BlockSpec((3, c, AUG), lambda hh: (hh, 0, 0)),
            pl.BlockSpec((3, 1, AUG), lambda hh: (hh, 0, 0)),
            pl.BlockSpec((3, c, AUG), lambda hh: (hh, 0, 0)),
            pl.BlockSpec((3, 1, AUG), lambda hh: (hh, 0, 0)),
            pl.BlockSpec((3, d, AUG), lambda hh: (hh, 0, 0)),
            pl.BlockSpec((3, 1, AUG), lambda hh: (hh, 0, 0)),
            pl.BlockSpec((3, c, vd), lambda hh: (hh, 0, 0)),
            pl.BlockSpec((3, 1, vd), lambda hh: (hh, 0, 0)),
            pl.BlockSpec((3, vd, c), lambda hh: (hh, 0, 0)),
            pl.BlockSpec((1, c), lambda hh: (0, 0)),
        ],
        out_specs=pl.BlockSpec((n, c), lambda hh: (0, 0)),
        out_shape=jax.ShapeDtypeStruct((n, c), _F32),
    )(feat2, pos2, mask2, wq_aug, bq_aug, wk_aug, bk_aug, wpos_aug, sel_aug,
      wv, bv, wo, bo)

    return out2.reshape(b, n, c)
